# baseline (speedup 1.0000x reference)
import functools
import jax
import jax.numpy as jnp
from jax import lax
from jax.experimental import pallas as pl
from jax.experimental.pallas import tpu as pltpu
from jax.experimental.pallas import tpu_sc as plsc

_NC = 2
_NS = 16
_NW = _NC * _NS
_L = 16

_CHUNK = 128
_SPAN = 40
_ACC_ROWS = 10240
_ROWS_PER_TILE = _ACC_ROWS // _NS
_DEG_ROWS = _ACC_ROWS // _L


def _deg_body(n_chunks, dstb_hbm, out_hbm, idx_d, bounce, acc_sh, sem):
    c = lax.axis_index("c")
    s = lax.axis_index("s")
    wid = c * _NS + s

    def fill(val):
        v16 = jnp.full((_L,), val, jnp.float32)

        def row(i, _):
            bounce[i] = v16
            return 0

        lax.fori_loop(0, _CHUNK, row, 0)

    fill(0.0)
    for r in range(_ROWS_PER_TILE // _CHUNK):
        pltpu.sync_copy(bounce, acc_sh.at[pl.ds(s * _ROWS_PER_TILE + r * _CHUNK, _CHUNK)])
    plsc.subcore_barrier()

    fill(1.0)
    pltpu.async_copy(dstb_hbm.at[wid], idx_d, sem).wait()

    def chunk(j, _):
        pltpu.sync_copy(bounce, acc_sh.at[idx_d.at[j]], add=True)
        return 0

    lax.fori_loop(0, n_chunks, chunk, 0)
    plsc.subcore_barrier()

    for r in range(_ROWS_PER_TILE // _CHUNK):
        off = s * _ROWS_PER_TILE + r * _CHUNK
        pltpu.sync_copy(acc_sh.at[pl.ds(off, _CHUNK)], bounce)
        pltpu.sync_copy(bounce, out_hbm.at[c, pl.ds(off, _CHUNK)])


def _agg_body(nsp0, nsp1, d, xs_hbm, srcb_hbm, dstb_hbm, out_hbm,
              idx_s, idx_d, rows0, rows1, acc_sh, gsem0, gsem1):
    c = lax.axis_index("c")
    s = lax.axis_index("s")
    z16 = jnp.zeros((_L,), jnp.float32)
    npair = _SPAN // 2

    def zero_row(i, _):
        def zero_seg(j, _2):
            rows0[i, pl.ds(j * _L, _L)] = z16
            return 0
        lax.fori_loop(0, d // _L, zero_seg, 0)
        return 0

    lax.fori_loop(0, _CHUNK, zero_row, 0)
    for r in range(_ROWS_PER_TILE // _CHUNK):
        pltpu.sync_copy(rows0, acc_sh.at[pl.ds(s * _ROWS_PER_TILE + r * _CHUNK, _CHUNK)])
    plsc.subcore_barrier()

    def gather(j, buf, sem):
        return pltpu.async_copy(xs_hbm.at[idx_s.at[j]], buf, sem)

    def gwait(j, buf, sem):
        pltpu.make_async_copy(xs_hbm.at[idx_s.at[j]], buf, sem).wait()

    def span(base_row):
        pltpu.async_copy(srcb_hbm.at[pl.ds(base_row, _SPAN)], idx_s, gsem0).wait()
        pltpu.async_copy(dstb_hbm.at[pl.ds(base_row, _SPAN)], idx_d, gsem0).wait()
        gather(0, rows0, gsem0)
        gather(1, rows1, gsem1)

        def pair(t, _):
            a = 2 * t
            b = 2 * t + 1
            gwait(a, rows0, gsem0)
            pltpu.sync_copy(rows0, acc_sh.at[idx_d.at[a]], add=True)

            @pl.when(t < npair - 1)
            def _pf0():
                gather(a + 2, rows0, gsem0)

            gwait(b, rows1, gsem1)
            pltpu.sync_copy(rows1, acc_sh.at[idx_d.at[b]], add=True)

            @pl.when(t < npair - 1)
            def _pf1():
                gather(b + 2, rows1, gsem1)

            return 0

        lax.fori_loop(0, npair, pair, 0)

    @pl.when(c == 0)
    def _sc0():
        for k in range(nsp0):
            span((s * nsp0 + k) * _SPAN)

    @pl.when(c == 1)
    def _sc1():
        for k in range(nsp1):
            span((_NS * nsp0 + s * nsp1 + k) * _SPAN)

    plsc.subcore_barrier()

    for r in range(_ROWS_PER_TILE // _CHUNK):
        off = s * _ROWS_PER_TILE + r * _CHUNK
        pltpu.sync_copy(acc_sh.at[pl.ds(off, _CHUNK)], rows0)
        pltpu.sync_copy(rows0, out_hbm.at[c, pl.ds(off, _CHUNK)])


def _sc_mesh():
    return plsc.VectorSubcoreMesh(core_axis_name="c", subcore_axis_name="s",
                                  num_cores=_NC, num_subcores=_NS)


def _run_deg(dstb, n_chunks):
    k = pl.kernel(
        functools.partial(_deg_body, n_chunks),
        out_type=jax.ShapeDtypeStruct((_NC, _ACC_ROWS, _L), jnp.float32),
        mesh=_sc_mesh(),
        scratch_types=[
            pltpu.VMEM((n_chunks, _CHUNK), jnp.int32),
            pltpu.VMEM((_CHUNK, _L), jnp.float32),
            pltpu.VMEM_SHARED((_ACC_ROWS, _L), jnp.float32),
            pltpu.SemaphoreType.DMA,
        ],
    )
    return k(dstb)


def _run_agg(xs, srcb, dstb, nsp0, nsp1, d):
    k = pl.kernel(
        functools.partial(_agg_body, nsp0, nsp1, d),
        out_type=jax.ShapeDtypeStruct((_NC, _ACC_ROWS, d), jnp.float32),
        mesh=_sc_mesh(),
        scratch_types=[
            pltpu.VMEM((_SPAN, _CHUNK), jnp.int32),
            pltpu.VMEM((_SPAN, _CHUNK), jnp.int32),
            pltpu.VMEM((_CHUNK, d), jnp.float32),
            pltpu.VMEM((_CHUNK, d), jnp.float32),
            pltpu.VMEM_SHARED((_ACC_ROWS, d), jnp.float32),
            pltpu.SemaphoreType.DMA,
            pltpu.SemaphoreType.DMA,
        ],
    )
    return k(xs, srcb, dstb)


def _prep_body(deg_ref, x_ref, dinv_ref, xs_ref):
    deg = jnp.sum(deg_ref[...], axis=0)[:, :1] + 1.0
    dinv = jax.lax.rsqrt(deg)
    dinv_ref[...] = dinv
    xs_ref[...] = x_ref[...] * dinv


def _run_prep(deg3, x):
    n = x.shape[0]
    nb = 10
    bs = n // nb
    return pl.pallas_call(
        _prep_body,
        grid=(nb,),
        in_specs=[
            pl.BlockSpec((_NC, bs, _L), lambda i: (0, i, 0)),
            pl.BlockSpec((bs, 128), lambda i: (i, 0)),
        ],
        out_specs=[
            pl.BlockSpec((bs, 1), lambda i: (i, 0)),
            pl.BlockSpec((bs, 128), lambda i: (i, 0)),
        ],
        out_shape=[
            jax.ShapeDtypeStruct((n, 1), jnp.float32),
            jax.ShapeDtypeStruct((n, 128), jnp.float32),
        ],
    )(deg3, x)


def _layer1_body(p0_ref, p1_ref, dinv_ref, x_ref, w1_ref, b1_ref, w2_ref,
                 h2_ref, hs2_ref):
    dinv = dinv_ref[...]
    t1 = dinv * (p0_ref[...] + p1_ref[...]) + (dinv * dinv) * x_ref[...]
    h = jnp.dot(t1, w1_ref[...], preferred_element_type=jnp.float32) + b1_ref[...]
    h = jnp.maximum(h, 0.0)
    h2 = jnp.dot(h, w2_ref[...], preferred_element_type=jnp.float32)
    h2_ref[...] = h2
    hs2_ref[...] = h2 * dinv


def _run_layer1(p0, p1, dinv, x, w1, b1, w2p):
    n = x.shape[0]
    nb = 10
    bs = n // nb
    dp = w2p.shape[1]
    return pl.pallas_call(
        _layer1_body,
        grid=(nb,),
        in_specs=[
            pl.BlockSpec((bs, 128), lambda i: (i, 0)),
            pl.BlockSpec((bs, 128), lambda i: (i, 0)),
            pl.BlockSpec((bs, 1), lambda i: (i, 0)),
            pl.BlockSpec((bs, 128), lambda i: (i, 0)),
            pl.BlockSpec((128, 128), lambda i: (0, 0)),
            pl.BlockSpec((1, 128), lambda i: (0, 0)),
            pl.BlockSpec((128, dp), lambda i: (0, 0)),
        ],
        out_specs=[
            pl.BlockSpec((bs, dp), lambda i: (i, 0)),
            pl.BlockSpec((bs, dp), lambda i: (i, 0)),
        ],
        out_shape=[
            jax.ShapeDtypeStruct((n, dp), jnp.float32),
            jax.ShapeDtypeStruct((n, dp), jnp.float32),
        ],
    )(p0, p1, dinv, x, w1, b1, w2p)


def _final_body(q0_ref, q1_ref, dinv_ref, h2_ref, b2_ref, out_ref):
    dinv = dinv_ref[...]
    out_ref[...] = (dinv * (q0_ref[...] + q1_ref[...])
                    + (dinv * dinv) * h2_ref[...] + b2_ref[...])


def _run_final(q0, q1, dinv, h2, b2p):
    n = h2.shape[0]
    nb = 10
    bs = n // nb
    dp = h2.shape[1]
    return pl.pallas_call(
        _final_body,
        grid=(nb,),
        in_specs=[
            pl.BlockSpec((bs, dp), lambda i: (i, 0)),
            pl.BlockSpec((bs, dp), lambda i: (i, 0)),
            pl.BlockSpec((bs, 1), lambda i: (i, 0)),
            pl.BlockSpec((bs, dp), lambda i: (i, 0)),
            pl.BlockSpec((1, dp), lambda i: (0, 0)),
        ],
        out_specs=pl.BlockSpec((bs, dp), lambda i: (i, 0)),
        out_shape=jax.ShapeDtypeStruct((n, dp), jnp.float32),
    )(q0, q1, dinv, h2, b2p)


def kernel(x, edge_index, W1, b1, W2, b2):
    n = x.shape[0]
    e = edge_index.shape[1]
    out_dim = W2.shape[1]
    dp = 128

    src = edge_index[0].astype(jnp.int32)
    dst = edge_index[1].astype(jnp.int32)

    nsp0, nsp1 = 1, 3
    tot_chunks = _NS * (nsp0 + nsp1) * _SPAN
    ep = tot_chunks * _CHUNK
    pad = ep - e
    src_p = jnp.concatenate([src, jnp.zeros((pad,), jnp.int32)])
    dst_p = jnp.concatenate([dst, jnp.full((pad,), n, jnp.int32)])
    srcb = src_p.reshape(tot_chunks, _CHUNK)
    dstb = dst_p.reshape(tot_chunks, _CHUNK)
    per_w = tot_chunks // _NW
    dstb3 = dst_p.reshape(_NW, per_w, _CHUNK)

    deg = _run_deg(dstb3, per_w)
    dinv, xs = _run_prep(deg[:, :n, :], x)

    p = _run_agg(xs, srcb, dstb, nsp0, nsp1, 128)
    p0 = p[0, :n, :]
    p1 = p[1, :n, :]

    w2p = jnp.pad(W2, ((0, 0), (0, dp - out_dim)))
    b1r = b1.reshape(1, 128)
    h2, hs2 = _run_layer1(p0, p1, dinv, x, W1, b1r, w2p)

    q = _run_agg(hs2, srcb, dstb, nsp0, nsp1, dp)
    q0 = q[0, :n, :]
    q1 = q[1, :n, :]

    b2p = jnp.pad(b2, (0, dp - out_dim)).reshape(1, dp)
    out48 = _run_final(q0, q1, dinv, h2, b2p)
    return (out48[:, :out_dim], None)


# speedup vs baseline: 1.1152x; 1.1152x over previous
import functools
import jax
import jax.numpy as jnp
from jax import lax
from jax.experimental import pallas as pl
from jax.experimental.pallas import tpu as pltpu
from jax.experimental.pallas import tpu_sc as plsc

_NC = 2
_NS = 16
_NW = _NC * _NS
_L = 16

_CHUNK = 128
_SPAN = 40
_ACC_ROWS = 10240
_ROWS_PER_TILE = _ACC_ROWS // _NS
_DEG_ROWS = _ACC_ROWS // _L


def _deg_body(n_chunks, dstb_hbm, out_hbm, idx_d, bounce, acc_sh, sem):
    c = lax.axis_index("c")
    s = lax.axis_index("s")
    wid = c * _NS + s

    def fill(val):
        v16 = jnp.full((_L,), val, jnp.float32)

        def row(i, _):
            bounce[i] = v16
            return 0

        lax.fori_loop(0, _CHUNK, row, 0)

    fill(0.0)
    for r in range(_ROWS_PER_TILE // _CHUNK):
        pltpu.sync_copy(bounce, acc_sh.at[pl.ds(s * _ROWS_PER_TILE + r * _CHUNK, _CHUNK)])
    plsc.subcore_barrier()

    fill(1.0)
    pltpu.async_copy(dstb_hbm.at[wid], idx_d, sem).wait()

    def chunk(j, _):
        pltpu.sync_copy(bounce, acc_sh.at[idx_d.at[j]], add=True)
        return 0

    lax.fori_loop(0, n_chunks, chunk, 0)
    plsc.subcore_barrier()

    for r in range(_ROWS_PER_TILE // _CHUNK):
        off = s * _ROWS_PER_TILE + r * _CHUNK
        pltpu.sync_copy(acc_sh.at[pl.ds(off, _CHUNK)], bounce)
        pltpu.sync_copy(bounce, out_hbm.at[c, pl.ds(off, _CHUNK)])


def _agg_body(nsp0, nsp1, d, xs_hbm, srcb_hbm, dstb_hbm, out_hbm,
              idx_s, idx_d, rows0, rows1, acc_sh, gsem0, gsem1):
    c = lax.axis_index("c")
    s = lax.axis_index("s")
    z16 = jnp.zeros((_L,), jnp.float32)
    npair = _SPAN // 2

    def zero_row(i, _):
        def zero_seg(j, _2):
            rows0[i, pl.ds(j * _L, _L)] = z16
            return 0
        lax.fori_loop(0, d // _L, zero_seg, 0)
        return 0

    lax.fori_loop(0, _CHUNK, zero_row, 0)
    for r in range(_ROWS_PER_TILE // _CHUNK):
        pltpu.sync_copy(rows0, acc_sh.at[pl.ds(s * _ROWS_PER_TILE + r * _CHUNK, _CHUNK)])
    plsc.subcore_barrier()

    def gather(j, buf, sem):
        return pltpu.async_copy(xs_hbm.at[idx_s.at[j]], buf, sem)

    def gwait(j, buf, sem):
        pltpu.make_async_copy(xs_hbm.at[idx_s.at[j]], buf, sem).wait()

    def span(base_row):
        pltpu.async_copy(srcb_hbm.at[pl.ds(base_row, _SPAN)], idx_s, gsem0).wait()
        pltpu.async_copy(dstb_hbm.at[pl.ds(base_row, _SPAN)], idx_d, gsem0).wait()
        gather(0, rows0, gsem0)
        gather(1, rows1, gsem1)

        def pair(t, _):
            a = 2 * t
            b = 2 * t + 1
            gwait(a, rows0, gsem0)
            pltpu.sync_copy(rows0, acc_sh.at[idx_d.at[a]], add=True)

            @pl.when(t < npair - 1)
            def _pf0():
                gather(a + 2, rows0, gsem0)

            gwait(b, rows1, gsem1)
            pltpu.sync_copy(rows1, acc_sh.at[idx_d.at[b]], add=True)

            @pl.when(t < npair - 1)
            def _pf1():
                gather(b + 2, rows1, gsem1)

            return 0

        lax.fori_loop(0, npair, pair, 0)

    @pl.when(c == 0)
    def _sc0():
        for k in range(nsp0):
            span((s * nsp0 + k) * _SPAN)

    @pl.when(c == 1)
    def _sc1():
        for k in range(nsp1):
            span((_NS * nsp0 + s * nsp1 + k) * _SPAN)

    plsc.subcore_barrier()

    for r in range(_ROWS_PER_TILE // _CHUNK):
        off = s * _ROWS_PER_TILE + r * _CHUNK
        pltpu.sync_copy(acc_sh.at[pl.ds(off, _CHUNK)], rows0)
        pltpu.sync_copy(rows0, out_hbm.at[c, pl.ds(off, _CHUNK)])


def _sc_mesh():
    return plsc.VectorSubcoreMesh(core_axis_name="c", subcore_axis_name="s",
                                  num_cores=_NC, num_subcores=_NS)


def _run_deg(dstb, n_chunks):
    k = pl.kernel(
        functools.partial(_deg_body, n_chunks),
        out_type=jax.ShapeDtypeStruct((_NC, _ACC_ROWS, _L), jnp.float32),
        mesh=_sc_mesh(),
        scratch_types=[
            pltpu.VMEM((n_chunks, _CHUNK), jnp.int32),
            pltpu.VMEM((_CHUNK, _L), jnp.float32),
            pltpu.VMEM_SHARED((_ACC_ROWS, _L), jnp.float32),
            pltpu.SemaphoreType.DMA,
        ],
    )
    return k(dstb)


def _run_agg(xs, srcb, dstb, nsp0, nsp1, d):
    k = pl.kernel(
        functools.partial(_agg_body, nsp0, nsp1, d),
        out_type=jax.ShapeDtypeStruct((_NC, _ACC_ROWS, d), jnp.float32),
        mesh=_sc_mesh(),
        scratch_types=[
            pltpu.VMEM((_SPAN, _CHUNK), jnp.int32),
            pltpu.VMEM((_SPAN, _CHUNK), jnp.int32),
            pltpu.VMEM((_CHUNK, d), jnp.float32),
            pltpu.VMEM((_CHUNK, d), jnp.float32),
            pltpu.VMEM_SHARED((_ACC_ROWS, d), jnp.float32),
            pltpu.SemaphoreType.DMA,
            pltpu.SemaphoreType.DMA,
        ],
    )
    return k(xs, srcb, dstb)


def _prep_body(deg_ref, x_ref, dinv_ref, xs_ref):
    deg = jnp.sum(deg_ref[...], axis=0)[:, :1] + 1.0
    dinv = jax.lax.rsqrt(deg)
    dinv_ref[...] = dinv
    xs_ref[...] = x_ref[...] * dinv


def _run_prep(deg3, x):
    n = x.shape[0]
    nb = 10
    bs = n // nb
    return pl.pallas_call(
        _prep_body,
        grid=(nb,),
        in_specs=[
            pl.BlockSpec((_NC, bs, _L), lambda i: (0, i, 0)),
            pl.BlockSpec((bs, 128), lambda i: (i, 0)),
        ],
        out_specs=[
            pl.BlockSpec((bs, 1), lambda i: (i, 0)),
            pl.BlockSpec((bs, 128), lambda i: (i, 0)),
        ],
        out_shape=[
            jax.ShapeDtypeStruct((n, 1), jnp.float32),
            jax.ShapeDtypeStruct((n, 128), jnp.float32),
        ],
    )(deg3, x)


def _layer1_body(p0_ref, p1_ref, dinv_ref, x_ref, w1_ref, b1_ref, w2_ref,
                 h2_ref, hs2_ref):
    dinv = dinv_ref[...]
    t1 = dinv * (p0_ref[...] + p1_ref[...]) + (dinv * dinv) * x_ref[...]
    h = jnp.dot(t1, w1_ref[...], preferred_element_type=jnp.float32) + b1_ref[...]
    h = jnp.maximum(h, 0.0)
    h2 = jnp.dot(h, w2_ref[...], preferred_element_type=jnp.float32)
    h2_ref[...] = h2
    hs2_ref[...] = h2 * dinv


def _run_layer1(p0, p1, dinv, x, w1, b1, w2p):
    n = x.shape[0]
    nb = 10
    bs = n // nb
    dp = w2p.shape[1]
    return pl.pallas_call(
        _layer1_body,
        grid=(nb,),
        in_specs=[
            pl.BlockSpec((bs, 128), lambda i: (i, 0)),
            pl.BlockSpec((bs, 128), lambda i: (i, 0)),
            pl.BlockSpec((bs, 1), lambda i: (i, 0)),
            pl.BlockSpec((bs, 128), lambda i: (i, 0)),
            pl.BlockSpec((128, 128), lambda i: (0, 0)),
            pl.BlockSpec((1, 128), lambda i: (0, 0)),
            pl.BlockSpec((128, dp), lambda i: (0, 0)),
        ],
        out_specs=[
            pl.BlockSpec((bs, dp), lambda i: (i, 0)),
            pl.BlockSpec((bs, dp), lambda i: (i, 0)),
        ],
        out_shape=[
            jax.ShapeDtypeStruct((n, dp), jnp.float32),
            jax.ShapeDtypeStruct((n, dp), jnp.float32),
        ],
    )(p0, p1, dinv, x, w1, b1, w2p)


def _final_body(q0_ref, q1_ref, dinv_ref, h2_ref, b2_ref, out_ref):
    dinv = dinv_ref[...]
    out_ref[...] = (dinv * (q0_ref[...] + q1_ref[...])
                    + (dinv * dinv) * h2_ref[...] + b2_ref[...])


def _run_final(q0, q1, dinv, h2, b2p):
    n = h2.shape[0]
    nb = 10
    bs = n // nb
    dp = h2.shape[1]
    return pl.pallas_call(
        _final_body,
        grid=(nb,),
        in_specs=[
            pl.BlockSpec((bs, dp), lambda i: (i, 0)),
            pl.BlockSpec((bs, dp), lambda i: (i, 0)),
            pl.BlockSpec((bs, 1), lambda i: (i, 0)),
            pl.BlockSpec((bs, dp), lambda i: (i, 0)),
            pl.BlockSpec((1, dp), lambda i: (0, 0)),
        ],
        out_specs=pl.BlockSpec((bs, dp), lambda i: (i, 0)),
        out_shape=jax.ShapeDtypeStruct((n, dp), jnp.float32),
    )(q0, q1, dinv, h2, b2p)


def kernel(x, edge_index, W1, b1, W2, b2):
    n = x.shape[0]
    e = edge_index.shape[1]
    out_dim = W2.shape[1]
    dp = 128

    src = edge_index[0].astype(jnp.int32)
    dst = edge_index[1].astype(jnp.int32)

    nsp0, nsp1 = 3, 1
    tot_chunks = _NS * (nsp0 + nsp1) * _SPAN
    ep = tot_chunks * _CHUNK
    pad = ep - e
    src_p = jnp.concatenate([src, jnp.zeros((pad,), jnp.int32)])
    dst_p = jnp.concatenate([dst, jnp.full((pad,), n, jnp.int32)])
    srcb = src_p.reshape(tot_chunks, _CHUNK)
    dstb = dst_p.reshape(tot_chunks, _CHUNK)
    per_w = tot_chunks // _NW
    dstb3 = dst_p.reshape(_NW, per_w, _CHUNK)

    deg = _run_deg(dstb3, per_w)
    dinv, xs = _run_prep(deg[:, :n, :], x)

    p = _run_agg(xs, srcb, dstb, nsp0, nsp1, 128)
    p0 = p[0, :n, :]
    p1 = p[1, :n, :]

    w2p = jnp.pad(W2, ((0, 0), (0, dp - out_dim)))
    b1r = b1.reshape(1, 128)
    h2, hs2 = _run_layer1(p0, p1, dinv, x, W1, b1r, w2p)

    q = _run_agg(hs2, srcb, dstb, nsp0, nsp1, dp)
    q0 = q[0, :n, :]
    q1 = q[1, :n, :]

    b2p = jnp.pad(b2, (0, dp - out_dim)).reshape(1, dp)
    out48 = _run_final(q0, q1, dinv, h2, b2p)
    return (out48[:, :out_dim], None)


# speedup vs baseline: 1.4976x; 1.3429x over previous
import functools
import jax
import jax.numpy as jnp
from jax import lax
from jax.experimental import pallas as pl
from jax.experimental.pallas import tpu as pltpu
from jax.experimental.pallas import tpu_sc as plsc

_NC = 2
_NS = 16
_NW = _NC * _NS
_L = 16

_CHUNK = 128
_ACC_ROWS = 10240
_ROWS_PER_TILE = _ACC_ROWS // _NS
_DEG_ROWS = _ACC_ROWS // _L


def _deg_body(n_chunks, dstb_hbm, out_hbm, idx_d, bounce, acc_sh, sem):
    c = lax.axis_index("c")
    s = lax.axis_index("s")
    wid = c * _NS + s

    def fill(val):
        v16 = jnp.full((_L,), val, jnp.float32)

        def row(i, _):
            bounce[i] = v16
            return 0

        lax.fori_loop(0, _CHUNK, row, 0)

    fill(0.0)
    for r in range(_ROWS_PER_TILE // _CHUNK):
        pltpu.sync_copy(bounce, acc_sh.at[pl.ds(s * _ROWS_PER_TILE + r * _CHUNK, _CHUNK)])
    plsc.subcore_barrier()

    fill(1.0)
    pltpu.async_copy(dstb_hbm.at[wid], idx_d, sem).wait()

    def chunk(j, _):
        pltpu.sync_copy(bounce, acc_sh.at[idx_d.at[j]], add=True)
        return 0

    lax.fori_loop(0, n_chunks, chunk, 0)
    plsc.subcore_barrier()

    for r in range(_ROWS_PER_TILE // _CHUNK):
        off = s * _ROWS_PER_TILE + r * _CHUNK
        pltpu.sync_copy(acc_sh.at[pl.ds(off, _CHUNK)], bounce)
        pltpu.sync_copy(bounce, out_hbm.at[c, pl.ds(off, _CHUNK)])


def _agg_body(n_chunks, d, xs_hbm, srcb_hbm, dstb_hbm, out_hbm,
              idx_s, idx_d, rows, acc_sh, sem):
    c = lax.axis_index("c")
    s = lax.axis_index("s")
    wid = c * _NS + s
    z16 = jnp.zeros((_L,), jnp.float32)

    def zero_row(i, _):
        def zero_seg(j, _2):
            rows[i, pl.ds(j * _L, _L)] = z16
            return 0
        lax.fori_loop(0, d // _L, zero_seg, 0)
        return 0

    lax.fori_loop(0, _CHUNK, zero_row, 0)
    for r in range(_ROWS_PER_TILE // _CHUNK):
        pltpu.sync_copy(rows, acc_sh.at[pl.ds(s * _ROWS_PER_TILE + r * _CHUNK, _CHUNK)])
    plsc.subcore_barrier()

    pltpu.async_copy(srcb_hbm.at[wid], idx_s, sem).wait()
    pltpu.async_copy(dstb_hbm.at[wid], idx_d, sem).wait()

    def chunk(j, _):
        pltpu.async_copy(xs_hbm.at[idx_s.at[j]], rows, sem).wait()
        pltpu.sync_copy(rows, acc_sh.at[idx_d.at[j]], add=True)
        return 0

    lax.fori_loop(0, n_chunks, chunk, 0)
    plsc.subcore_barrier()

    for r in range(_ROWS_PER_TILE // _CHUNK):
        off = s * _ROWS_PER_TILE + r * _CHUNK
        pltpu.sync_copy(acc_sh.at[pl.ds(off, _CHUNK)], rows)
        pltpu.sync_copy(rows, out_hbm.at[c, pl.ds(off, _CHUNK)])


def _sc_mesh():
    return plsc.VectorSubcoreMesh(core_axis_name="c", subcore_axis_name="s",
                                  num_cores=_NC, num_subcores=_NS)


def _run_deg(dstb, n_chunks):
    k = pl.kernel(
        functools.partial(_deg_body, n_chunks),
        out_type=jax.ShapeDtypeStruct((_NC, _ACC_ROWS, _L), jnp.float32),
        mesh=_sc_mesh(),
        scratch_types=[
            pltpu.VMEM((n_chunks, _CHUNK), jnp.int32),
            pltpu.VMEM((_CHUNK, _L), jnp.float32),
            pltpu.VMEM_SHARED((_ACC_ROWS, _L), jnp.float32),
            pltpu.SemaphoreType.DMA,
        ],
    )
    return k(dstb)


def _run_agg(xs, srcb, dstb, n_chunks, d):
    k = pl.kernel(
        functools.partial(_agg_body, n_chunks, d),
        out_type=jax.ShapeDtypeStruct((_NC, _ACC_ROWS, d), jnp.float32),
        mesh=_sc_mesh(),
        scratch_types=[
            pltpu.VMEM((n_chunks, _CHUNK), jnp.int32),
            pltpu.VMEM((n_chunks, _CHUNK), jnp.int32),
            pltpu.VMEM((_CHUNK, d), jnp.float32),
            pltpu.VMEM_SHARED((_ACC_ROWS, d), jnp.float32),
            pltpu.SemaphoreType.DMA,
        ],
    )
    return k(xs, srcb, dstb)


def _prep_body(deg_ref, x_ref, dinv_ref, xs_ref):
    deg = jnp.sum(deg_ref[...], axis=0)[:, :1] + 1.0
    dinv = jax.lax.rsqrt(deg)
    dinv_ref[...] = dinv
    xs_ref[...] = x_ref[...] * dinv


def _run_prep(deg3, x):
    n = x.shape[0]
    nb = 10
    bs = n // nb
    return pl.pallas_call(
        _prep_body,
        grid=(nb,),
        in_specs=[
            pl.BlockSpec((_NC, bs, _L), lambda i: (0, i, 0)),
            pl.BlockSpec((bs, 128), lambda i: (i, 0)),
        ],
        out_specs=[
            pl.BlockSpec((bs, 1), lambda i: (i, 0)),
            pl.BlockSpec((bs, 128), lambda i: (i, 0)),
        ],
        out_shape=[
            jax.ShapeDtypeStruct((n, 1), jnp.float32),
            jax.ShapeDtypeStruct((n, 128), jnp.float32),
        ],
    )(deg3, x)


def _layer1_body(p0_ref, p1_ref, dinv_ref, x_ref, w1_ref, b1_ref, w2_ref,
                 h2_ref, hs2_ref):
    dinv = dinv_ref[...]
    t1 = dinv * (p0_ref[...] + p1_ref[...]) + (dinv * dinv) * x_ref[...]
    h = jnp.dot(t1, w1_ref[...], preferred_element_type=jnp.float32) + b1_ref[...]
    h = jnp.maximum(h, 0.0)
    h2 = jnp.dot(h, w2_ref[...], preferred_element_type=jnp.float32)
    h2_ref[...] = h2
    hs2_ref[...] = h2 * dinv


def _run_layer1(p0, p1, dinv, x, w1, b1, w2p):
    n = x.shape[0]
    nb = 10
    bs = n // nb
    dp = w2p.shape[1]
    return pl.pallas_call(
        _layer1_body,
        grid=(nb,),
        in_specs=[
            pl.BlockSpec((bs, 128), lambda i: (i, 0)),
            pl.BlockSpec((bs, 128), lambda i: (i, 0)),
            pl.BlockSpec((bs, 1), lambda i: (i, 0)),
            pl.BlockSpec((bs, 128), lambda i: (i, 0)),
            pl.BlockSpec((128, 128), lambda i: (0, 0)),
            pl.BlockSpec((1, 128), lambda i: (0, 0)),
            pl.BlockSpec((128, dp), lambda i: (0, 0)),
        ],
        out_specs=[
            pl.BlockSpec((bs, dp), lambda i: (i, 0)),
            pl.BlockSpec((bs, dp), lambda i: (i, 0)),
        ],
        out_shape=[
            jax.ShapeDtypeStruct((n, dp), jnp.float32),
            jax.ShapeDtypeStruct((n, dp), jnp.float32),
        ],
    )(p0, p1, dinv, x, w1, b1, w2p)


def _final_body(q0_ref, q1_ref, dinv_ref, h2_ref, b2_ref, out_ref):
    dinv = dinv_ref[...]
    out_ref[...] = (dinv * (q0_ref[...] + q1_ref[...])
                    + (dinv * dinv) * h2_ref[...] + b2_ref[...])


def _run_final(q0, q1, dinv, h2, b2p):
    n = h2.shape[0]
    nb = 10
    bs = n // nb
    dp = h2.shape[1]
    return pl.pallas_call(
        _final_body,
        grid=(nb,),
        in_specs=[
            pl.BlockSpec((bs, dp), lambda i: (i, 0)),
            pl.BlockSpec((bs, dp), lambda i: (i, 0)),
            pl.BlockSpec((bs, 1), lambda i: (i, 0)),
            pl.BlockSpec((bs, dp), lambda i: (i, 0)),
            pl.BlockSpec((1, dp), lambda i: (0, 0)),
        ],
        out_specs=pl.BlockSpec((bs, dp), lambda i: (i, 0)),
        out_shape=jax.ShapeDtypeStruct((n, dp), jnp.float32),
    )(q0, q1, dinv, h2, b2p)


def kernel(x, edge_index, W1, b1, W2, b2):
    n = x.shape[0]
    e = edge_index.shape[1]
    out_dim = W2.shape[1]
    dp = 128

    src = edge_index[0].astype(jnp.int32)
    dst = edge_index[1].astype(jnp.int32)

    n_chunks_total = -(-e // _CHUNK)
    per_w = -(-n_chunks_total // _NW)
    ep = per_w * _NW * _CHUNK
    pad = ep - e
    src_p = jnp.concatenate([src, jnp.zeros((pad,), jnp.int32)])
    dst_p = jnp.concatenate([dst, jnp.full((pad,), n, jnp.int32)])
    srcb = src_p.reshape(_NW, per_w, _CHUNK)
    dstb = dst_p.reshape(_NW, per_w, _CHUNK)

    deg = _run_deg(dstb, per_w)
    dinv, xs = _run_prep(deg[:, :n, :], x)

    p = _run_agg(xs, srcb, dstb, per_w, 128)
    p0 = p[0, :n, :]
    p1 = p[1, :n, :]

    w2p = jnp.pad(W2, ((0, 0), (0, dp - out_dim)))
    b1r = b1.reshape(1, 128)
    h2, hs2 = _run_layer1(p0, p1, dinv, x, W1, b1r, w2p)

    q = _run_agg(hs2, srcb, dstb, per_w, dp)
    q0 = q[0, :n, :]
    q1 = q[1, :n, :]

    b2p = jnp.pad(b2, (0, dp - out_dim)).reshape(1, dp)
    out48 = _run_final(q0, q1, dinv, h2, b2p)
    return (out48[:, :out_dim], None)
